# baseline (speedup 1.0000x reference)
import functools

import jax
import jax.numpy as jnp
from jax import lax
from jax.experimental import pallas as pl
from jax.experimental.pallas import tpu as pltpu
from jax.experimental.pallas import tpu_sc as plsc

N = 1_600_000
B = 512
NELEM = 118
NC, NS, L = 2, 16, 16
NW = NC * NS
CHUNK = N // NW
BLK = 10_000
NBLK = CHUNK // BLK
ITERS = BLK // L
UNROLL = 5
ROWS = B // L
STRIDE = 17
ACCW = B * STRIDE + L

_mesh = plsc.VectorSubcoreMesh(
    core_axis_name="c", subcore_axis_name="s", num_cores=NC, num_subcores=NS
)


@functools.partial(
    pl.kernel,
    out_type=jax.ShapeDtypeStruct((NC, ROWS, L), jnp.float32),
    mesh=_mesh,
    compiler_params=pltpu.CompilerParams(needs_layout_passes=False),
    scratch_types=[
        pltpu.VMEM((BLK,), jnp.float32),
        pltpu.VMEM((BLK,), jnp.float32),
        pltpu.VMEM((BLK,), jnp.int32),
        pltpu.VMEM((BLK,), jnp.int32),
        pltpu.VMEM((BLK,), jnp.int32),
        pltpu.VMEM((BLK,), jnp.int32),
        pltpu.VMEM((ACCW,), jnp.float32),
        pltpu.VMEM((ACCW,), jnp.float32),
        pltpu.VMEM((ACCW,), jnp.float32),
        pltpu.VMEM((ACCW,), jnp.float32),
        pltpu.VMEM((ACCW,), jnp.float32),
        pltpu.VMEM((ROWS, L), jnp.float32),
        pltpu.VMEM((128,), jnp.float32),
        pltpu.VMEM((L,), jnp.float32),
        pltpu.VMEM((L,), jnp.float32),
        pltpu.VMEM((2, L), jnp.float32),
        pltpu.VMEM_SHARED((NS, ROWS, L), jnp.float32),
        pltpu.SemaphoreType.DMA,
        pltpu.SemaphoreType.DMA,
    ],
)
def _seg_kernel(off_hbm, typ_hbm, bid_hbm, bias_hbm, std_hbm, tot_hbm, out_hbm,
                off0, off1, typ0, typ1, bid0, bid1,
                acc0, acc1, acc2, acc3, acc4, loc,
                biasv, stdv, totv, fbuf, shared, sem0, sem1):
    accs = [acc0, acc1, acc2, acc3, acc4]
    c = lax.axis_index("c")
    s = lax.axis_index("s")
    w = c * NS + s
    bufs = [(off0, typ0, bid0, sem0), (off1, typ1, bid1, sem1)]

    def start_block(b):
        base = pl.multiple_of(w * CHUNK + b * BLK, 16)
        ob, tb, bb, sem = bufs[b % 2]
        return [
            pltpu.async_copy(off_hbm.at[pl.ds(base, BLK)], ob, sem),
            pltpu.async_copy(typ_hbm.at[pl.ds(base, BLK)], tb, sem),
            pltpu.async_copy(bid_hbm.at[pl.ds(base, BLK)], bb, sem),
        ]

    handles = start_block(0)

    pltpu.sync_copy(bias_hbm, biasv)
    pltpu.sync_copy(std_hbm, stdv.at[pl.ds(0, 1)])
    pltpu.sync_copy(tot_hbm, totv.at[pl.ds(0, 1)])

    zeros = jnp.zeros((L,), jnp.float32)
    iv = lax.iota(jnp.int32, L)

    def zacc(i, carry):
        for a in accs:
            a[pl.ds(i * L, L)] = zeros
        return carry
    lax.fori_loop(0, ACCW // L, zacc, 0)

    std_s = stdv[...][0]
    lane17 = iv * STRIDE

    for b in range(NBLK):
        nxt = start_block(b + 1) if b + 1 < NBLK else None
        for h in handles:
            h.wait()
        ob, tb, bb, _sem = bufs[b % 2]

        def body(i, carry):
            base = i * (UNROLL * L)
            for u, a in enumerate(accs):
                o = ob[pl.ds(base + u * L, L)]
                t = tb[pl.ds(base + u * L, L)]
                bi = bb[pl.ds(base + u * L, L)]
                bias = plsc.load_gather(biasv, [t])
                e = o * std_s + bias
                plsc.addupdate_scatter(a, [bi * STRIDE + iv], e)
            return carry
        lax.fori_loop(0, ITERS // UNROLL, body, 0)
        handles = nxt

    for j in range(ROWS):
        v = zeros
        for a in accs:
            for l in range(L):
                v = v + plsc.load_gather(a, [lane17 + (j * L * STRIDE + l)])
        loc[j, :] = v

    pltpu.sync_copy(loc, shared.at[s])
    plsc.subcore_barrier()

    tot_v = zeros + totv[...][0] * jnp.where(c == 0, 1.0, 0.0)
    v0 = tot_v
    v1 = tot_v
    for t in range(NS):
        pltpu.sync_copy(shared.at[t, pl.ds(2 * s, 2)], fbuf)
        v0 = v0 + fbuf[0, :]
        v1 = v1 + fbuf[1, :]
    fbuf[0, :] = v0
    fbuf[1, :] = v1
    pltpu.sync_copy(fbuf, out_hbm.at[c, pl.ds(2 * s, 2)])


def kernel(atomic_offset_energy, atom_type, batch_ids, cell,
           potential_bias, potential_std, potential_total):
    bias128 = jnp.zeros((128,), jnp.float32).at[:NELEM].set(potential_bias)
    partial = _seg_kernel(atomic_offset_energy, atom_type, batch_ids,
                          bias128, potential_std, potential_total)
    p = partial.reshape(NC, B)
    return p[0] + p[1]


# speedup vs baseline: 1.2053x; 1.2053x over previous
import functools

import jax
import jax.numpy as jnp
from jax import lax
from jax.experimental import pallas as pl
from jax.experimental.pallas import tpu as pltpu
from jax.experimental.pallas import tpu_sc as plsc

N = 1_600_000
B = 512
NELEM = 118
NC, NS, L = 2, 16, 16
NW = NC * NS
CHUNK = N // NW
BLK = 10_000
NBLK = CHUNK // BLK
ITERS = BLK // L
UNROLL = 5
ROWS = B // L
STRIDE = 17
ACCW = B * STRIDE + L

_mesh = plsc.VectorSubcoreMesh(
    core_axis_name="c", subcore_axis_name="s", num_cores=NC, num_subcores=NS
)


@functools.partial(
    pl.kernel,
    out_type=jax.ShapeDtypeStruct((NC, B), jnp.float32),
    mesh=_mesh,
    compiler_params=pltpu.CompilerParams(needs_layout_passes=False),
    scratch_types=[
        pltpu.VMEM((BLK,), jnp.float32),
        pltpu.VMEM((BLK,), jnp.float32),
        pltpu.VMEM((BLK,), jnp.int32),
        pltpu.VMEM((BLK,), jnp.int32),
        pltpu.VMEM((BLK,), jnp.int32),
        pltpu.VMEM((BLK,), jnp.int32),
        pltpu.VMEM((ACCW,), jnp.float32),
        pltpu.VMEM((ACCW,), jnp.float32),
        pltpu.VMEM((ACCW,), jnp.float32),
        pltpu.VMEM((ACCW,), jnp.float32),
        pltpu.VMEM((ACCW,), jnp.float32),
        pltpu.VMEM((B,), jnp.float32),
        pltpu.VMEM((128,), jnp.float32),
        pltpu.VMEM((L,), jnp.float32),
        pltpu.VMEM((L,), jnp.float32),
        pltpu.VMEM((2 * L,), jnp.float32),
        pltpu.VMEM_SHARED((NS, B), jnp.float32),
        pltpu.SemaphoreType.DMA,
        pltpu.SemaphoreType.DMA,
    ],
)
def _seg_kernel(off_hbm, typ_hbm, bid_hbm, bias_hbm, std_hbm, tot_hbm, out_hbm,
                off0, off1, typ0, typ1, bid0, bid1,
                acc0, acc1, acc2, acc3, acc4, loc,
                biasv, stdv, totv, fbuf, shared, sem0, sem1):
    accs = [acc0, acc1, acc2, acc3, acc4]
    c = lax.axis_index("c")
    s = lax.axis_index("s")
    w = c * NS + s
    bufs = [(off0, typ0, bid0, sem0), (off1, typ1, bid1, sem1)]

    def start_block(b):
        base = pl.multiple_of(w * CHUNK + b * BLK, 16)
        ob, tb, bb, sem = bufs[b % 2]
        return [
            pltpu.async_copy(off_hbm.at[pl.ds(base, BLK)], ob, sem),
            pltpu.async_copy(typ_hbm.at[pl.ds(base, BLK)], tb, sem),
            pltpu.async_copy(bid_hbm.at[pl.ds(base, BLK)], bb, sem),
        ]

    handles = start_block(0)

    pltpu.sync_copy(bias_hbm, biasv)
    pltpu.sync_copy(std_hbm, stdv.at[pl.ds(0, 1)])
    pltpu.sync_copy(tot_hbm, totv.at[pl.ds(0, 1)])

    zeros = jnp.zeros((L,), jnp.float32)
    iv = lax.iota(jnp.int32, L)

    def zacc(i, carry):
        for a in accs:
            a[pl.ds(i * L, L)] = zeros
        return carry
    lax.fori_loop(0, ACCW // L, zacc, 0)

    std_s = stdv[...][0]
    lane17 = iv * STRIDE

    for b in range(NBLK):
        nxt = start_block(b + 1) if b + 1 < NBLK else None
        for h in handles:
            h.wait()
        ob, tb, bb, _sem = bufs[b % 2]

        def body(i, carry):
            base = i * (UNROLL * L)
            for u, a in enumerate(accs):
                o = ob[pl.ds(base + u * L, L)]
                t = tb[pl.ds(base + u * L, L)]
                bi = bb[pl.ds(base + u * L, L)]
                bias = plsc.load_gather(biasv, [t])
                e = o * std_s + bias
                plsc.addupdate_scatter(a, [bi * STRIDE + iv], e)
            return carry
        lax.fori_loop(0, ITERS // UNROLL, body, 0)
        handles = nxt

    def reduce_row(j, carry):
        v = zeros
        for a in accs:
            for l in range(L):
                v = v + plsc.load_gather(a, [lane17 + (j * (L * STRIDE) + l)])
        loc[pl.ds(j * L, L)] = v
        return carry
    lax.fori_loop(0, ROWS, reduce_row, 0)

    pltpu.sync_copy(loc, shared.at[s])
    plsc.subcore_barrier()

    tot_v = zeros + totv[...][0] * jnp.where(c == 0, 1.0, 0.0)
    v0 = tot_v
    v1 = tot_v
    for t in range(NS):
        pltpu.sync_copy(shared.at[t, pl.ds(2 * L * s, 2 * L)], fbuf)
        v0 = v0 + fbuf[pl.ds(0, L)]
        v1 = v1 + fbuf[pl.ds(L, L)]
    fbuf[pl.ds(0, L)] = v0
    fbuf[pl.ds(L, L)] = v1
    pltpu.sync_copy(fbuf, out_hbm.at[c, pl.ds(2 * L * s, 2 * L)])


def kernel(atomic_offset_energy, atom_type, batch_ids, cell,
           potential_bias, potential_std, potential_total):
    bias128 = jnp.zeros((128,), jnp.float32).at[:NELEM].set(potential_bias)
    partial = _seg_kernel(atomic_offset_energy, atom_type, batch_ids,
                          bias128, potential_std, potential_total)
    return partial[0] + partial[1]


# speedup vs baseline: 1.2334x; 1.0233x over previous
import functools

import jax
import jax.numpy as jnp
from jax import lax
from jax.experimental import pallas as pl
from jax.experimental.pallas import tpu as pltpu
from jax.experimental.pallas import tpu_sc as plsc

N = 1_600_000
B = 512
NELEM = 118
NC, NS, L = 2, 16, 16
NW = NC * NS
CHUNK = N // NW
BLK = 10_000
NBLK = CHUNK // BLK
ITERS = BLK // L
UNROLL = 5
ROWS = B // L
STRIDE = 17
ACCW = B * STRIDE + L

_mesh = plsc.VectorSubcoreMesh(
    core_axis_name="c", subcore_axis_name="s", num_cores=NC, num_subcores=NS
)


@functools.partial(
    pl.kernel,
    out_type=jax.ShapeDtypeStruct((NC, B), jnp.float32),
    mesh=_mesh,
    compiler_params=pltpu.CompilerParams(needs_layout_passes=False),
    scratch_types=[
        pltpu.VMEM((BLK,), jnp.float32),
        pltpu.VMEM((BLK,), jnp.float32),
        pltpu.VMEM((BLK,), jnp.int32),
        pltpu.VMEM((BLK,), jnp.int32),
        pltpu.VMEM((BLK,), jnp.int32),
        pltpu.VMEM((BLK,), jnp.int32),
        pltpu.VMEM((ACCW,), jnp.float32),
        pltpu.VMEM((ACCW,), jnp.float32),
        pltpu.VMEM((ACCW,), jnp.float32),
        pltpu.VMEM((ACCW,), jnp.float32),
        pltpu.VMEM((ACCW,), jnp.float32),
        pltpu.VMEM((B,), jnp.float32),
        pltpu.VMEM((128,), jnp.float32),
        pltpu.VMEM((L,), jnp.float32),
        pltpu.VMEM((L,), jnp.float32),
        pltpu.VMEM((2 * L,), jnp.float32),
        pltpu.VMEM((NS, 2 * L), jnp.float32),
        pltpu.VMEM_SHARED((NS, B), jnp.float32),
        pltpu.SemaphoreType.DMA,
        pltpu.SemaphoreType.DMA,
        pltpu.SemaphoreType.DMA,
    ],
)
def _seg_kernel(off_hbm, typ_hbm, bid_hbm, bias_hbm, std_hbm, tot_hbm, out_hbm,
                off0, off1, typ0, typ1, bid0, bid1,
                acc0, acc1, acc2, acc3, acc4, loc,
                biasv, stdv, totv, fbuf, mbuf, shared, sem0, sem1, sem2):
    accs = [acc0, acc1, acc2, acc3, acc4]
    c = lax.axis_index("c")
    s = lax.axis_index("s")
    w = c * NS + s
    bufs = [(off0, typ0, bid0, sem0), (off1, typ1, bid1, sem1)]

    def start_block(b):
        base = pl.multiple_of(w * CHUNK + b * BLK, 16)
        ob, tb, bb, sem = bufs[b % 2]
        return [
            pltpu.async_copy(off_hbm.at[pl.ds(base, BLK)], ob, sem),
            pltpu.async_copy(typ_hbm.at[pl.ds(base, BLK)], tb, sem),
            pltpu.async_copy(bid_hbm.at[pl.ds(base, BLK)], bb, sem),
        ]

    handles = start_block(0)

    pltpu.sync_copy(bias_hbm, biasv)
    pltpu.sync_copy(std_hbm, stdv.at[pl.ds(0, 1)])
    pltpu.sync_copy(tot_hbm, totv.at[pl.ds(0, 1)])

    zeros = jnp.zeros((L,), jnp.float32)
    iv = lax.iota(jnp.int32, L)

    def zacc(i, carry):
        for a in accs:
            a[pl.ds(i * L, L)] = zeros
        return carry
    lax.fori_loop(0, ACCW // L, zacc, 0)

    std_s = stdv[...][0]
    lane17 = iv * STRIDE

    for b in range(NBLK):
        nxt = start_block(b + 1) if b + 1 < NBLK else None
        for h in handles:
            h.wait()
        ob, tb, bb, _sem = bufs[b % 2]

        def body(i, carry):
            base = i * (UNROLL * L)
            for u, a in enumerate(accs):
                o = ob[pl.ds(base + u * L, L)]
                t = tb[pl.ds(base + u * L, L)]
                bi = bb[pl.ds(base + u * L, L)]
                bias = plsc.load_gather(biasv, [t])
                e = o * std_s + bias
                plsc.addupdate_scatter(a, [bi * STRIDE + iv], e)
            return carry
        lax.fori_loop(0, ITERS // UNROLL, body, 0)
        handles = nxt

    def reduce_row(j, carry):
        v = zeros
        for a in accs:
            for l in range(L):
                v = v + plsc.load_gather(a, [lane17 + (j * (L * STRIDE) + l)])
        loc[pl.ds(j * L, L)] = v
        return carry
    lax.fori_loop(0, ROWS, reduce_row, 0)

    pltpu.sync_copy(loc, shared.at[s])
    plsc.subcore_barrier()

    merge = [
        pltpu.async_copy(shared.at[t, pl.ds(2 * L * s, 2 * L)], mbuf.at[t], sem2)
        for t in range(NS)
    ]
    for h in merge:
        h.wait()
    tot_v = zeros + totv[...][0] * jnp.where(c == 0, 1.0, 0.0)
    v0 = tot_v
    v1 = tot_v
    for t in range(NS):
        v0 = v0 + mbuf[t, pl.ds(0, L)]
        v1 = v1 + mbuf[t, pl.ds(L, L)]
    fbuf[pl.ds(0, L)] = v0
    fbuf[pl.ds(L, L)] = v1
    pltpu.sync_copy(fbuf, out_hbm.at[c, pl.ds(2 * L * s, 2 * L)])


def kernel(atomic_offset_energy, atom_type, batch_ids, cell,
           potential_bias, potential_std, potential_total):
    bias128 = jnp.zeros((128,), jnp.float32).at[:NELEM].set(potential_bias)
    partial = _seg_kernel(atomic_offset_energy, atom_type, batch_ids,
                          bias128, potential_std, potential_total)
    return partial[0] + partial[1]
